# baseline (speedup 1.0000x reference)
import functools

import jax
import jax.numpy as jnp
from jax.experimental import pallas as pl
from jax.experimental.pallas import tpu as pltpu

_NW = 25


def _fused_body(x_ref, w_ref, adj_ref, b_ref, out_ref, sup_ref):
    m = pl.program_id(0)

    @pl.when(m < _NW)
    def _support_chunk():
        chunk = x_ref.shape[0]
        x = x_ref[...].astype(jnp.bfloat16)
        w = w_ref[...].astype(jnp.bfloat16)
        sup_ref[pl.ds(m * chunk, chunk), :] = jnp.dot(
            x, w, preferred_element_type=jnp.float32
        ).astype(jnp.bfloat16)

    @pl.when(m >= _NW)
    def _spmm():
        a = adj_ref[...].astype(jnp.bfloat16)
        part = jnp.dot(a, sup_ref[...], preferred_element_type=jnp.float32)
        out_ref[...] = part + b_ref[...]


@functools.partial(jax.jit, static_argnames=())
def kernel(input, adj, W, b):
    n, in_f = input.shape
    out_f = W.shape[1]

    chunk = n // _NW
    bm = 512
    n_spmm = pl.cdiv(n, bm)
    b2 = b.reshape(1, out_f)

    out = pl.pallas_call(
        _fused_body,
        grid=(_NW + n_spmm,),
        in_specs=[
            pl.BlockSpec((chunk, in_f), lambda m: (jnp.minimum(m, _NW - 1), 0)),
            pl.BlockSpec((in_f, out_f), lambda m: (0, 0)),
            pl.BlockSpec((bm, n), lambda m: (jnp.maximum(m - _NW, 0), 0)),
            pl.BlockSpec((1, out_f), lambda m: (0, 0)),
        ],
        out_specs=pl.BlockSpec((bm, out_f), lambda m: (jnp.maximum(m - _NW, 0), 0)),
        out_shape=jax.ShapeDtypeStruct((n, out_f), jnp.float32),
        scratch_shapes=[pltpu.VMEM((n, out_f), jnp.bfloat16)],
        compiler_params=pltpu.CompilerParams(
            dimension_semantics=("arbitrary",),
        ),
    )(input, W, adj, b2)
    return out


# speedup vs baseline: 1.0656x; 1.0656x over previous
import functools

import jax
import jax.numpy as jnp
from jax.experimental import pallas as pl
from jax.experimental.pallas import tpu as pltpu

_NW = 10


def _fused_body(x_ref, w_ref, adj_hbm, b_ref, out_ref, sup_ref, abuf, sem):
    m = pl.program_id(0)
    bm = abuf.shape[1]
    n_blocks = adj_hbm.shape[0] // bm

    @pl.when(m == 0)
    def _start_first_slab():
        pltpu.make_async_copy(
            adj_hbm.at[pl.ds(0, bm), :], abuf.at[0], sem.at[0]
        ).start()

    @pl.when(m < _NW)
    def _support_chunk():
        chunk = x_ref.shape[0]
        x = x_ref[...].astype(jnp.bfloat16)
        w = w_ref[...].astype(jnp.bfloat16)
        sup_ref[pl.ds(m * chunk, chunk), :] = jnp.dot(
            x, w, preferred_element_type=jnp.float32
        ).astype(jnp.bfloat16)

    @pl.when(m >= _NW)
    def _spmm():
        i = m - _NW
        slot = jax.lax.rem(i, 2)
        nslot = jax.lax.rem(i + 1, 2)

        @pl.when(i + 1 < n_blocks)
        def _prefetch_next():
            pltpu.make_async_copy(
                adj_hbm.at[pl.ds((i + 1) * bm, bm), :], abuf.at[nslot], sem.at[nslot]
            ).start()

        pltpu.make_async_copy(
            adj_hbm.at[pl.ds(i * bm, bm), :], abuf.at[slot], sem.at[slot]
        ).wait()
        a = abuf[slot].astype(jnp.bfloat16)
        part = jnp.dot(a, sup_ref[...], preferred_element_type=jnp.float32)
        out_ref[...] = part + b_ref[...]


@functools.partial(jax.jit, static_argnames=())
def kernel(input, adj, W, b):
    n, in_f = input.shape
    out_f = W.shape[1]

    chunk = n // _NW
    bm = 400 if n % 400 == 0 else n
    n_blocks = n // bm
    b2 = b.reshape(1, out_f)

    out = pl.pallas_call(
        _fused_body,
        grid=(_NW + n_blocks,),
        in_specs=[
            pl.BlockSpec((chunk, in_f), lambda m: (jnp.minimum(m, _NW - 1), 0)),
            pl.BlockSpec((in_f, out_f), lambda m: (0, 0)),
            pl.BlockSpec(memory_space=pl.ANY),
            pl.BlockSpec((1, out_f), lambda m: (0, 0)),
        ],
        out_specs=pl.BlockSpec(
            (bm, out_f), lambda m: (jnp.maximum(m - _NW, 0), 0)
        ),
        out_shape=jax.ShapeDtypeStruct((n, out_f), jnp.float32),
        scratch_shapes=[
            pltpu.VMEM((n, out_f), jnp.bfloat16),
            pltpu.VMEM((2, bm, n), jnp.float32),
            pltpu.SemaphoreType.DMA((2,)),
        ],
        compiler_params=pltpu.CompilerParams(
            dimension_semantics=("arbitrary",),
        ),
    )(input, W, adj, b2)
    return out


# speedup vs baseline: 1.0818x; 1.0152x over previous
import functools

import jax
import jax.numpy as jnp
from jax.experimental import pallas as pl
from jax.experimental.pallas import tpu as pltpu

_NW = 5


def _fused_body(x_ref, w_ref, adj_hbm, b_ref, out_ref, sup_ref, abuf, sem):
    m = pl.program_id(0)
    bm = abuf.shape[1]
    n_blocks = adj_hbm.shape[0] // bm

    @pl.when(m == 0)
    def _start_first_slab():
        pltpu.make_async_copy(
            adj_hbm.at[pl.ds(0, bm), :], abuf.at[0], sem.at[0]
        ).start()

    @pl.when(m < _NW)
    def _support_chunk():
        chunk = x_ref.shape[0]
        x = x_ref[...].astype(jnp.bfloat16)
        w = w_ref[...].astype(jnp.bfloat16)
        sup_ref[pl.ds(m * chunk, chunk), :] = jnp.dot(
            x, w, preferred_element_type=jnp.float32
        ).astype(jnp.bfloat16)

    @pl.when(m >= _NW)
    def _spmm():
        i = m - _NW
        slot = jax.lax.rem(i, 2)
        nslot = jax.lax.rem(i + 1, 2)

        @pl.when(i + 1 < n_blocks)
        def _prefetch_next():
            pltpu.make_async_copy(
                adj_hbm.at[pl.ds((i + 1) * bm, bm), :], abuf.at[nslot], sem.at[nslot]
            ).start()

        pltpu.make_async_copy(
            adj_hbm.at[pl.ds(i * bm, bm), :], abuf.at[slot], sem.at[slot]
        ).wait()
        a = abuf[slot].astype(jnp.bfloat16)
        part = jnp.dot(a, sup_ref[...], preferred_element_type=jnp.float32)
        out_ref[...] = part + b_ref[...]


@functools.partial(jax.jit, static_argnames=())
def kernel(input, adj, W, b):
    n, in_f = input.shape
    out_f = W.shape[1]

    chunk = n // _NW
    bm = 400 if n % 400 == 0 else n
    n_blocks = n // bm
    b2 = b.reshape(1, out_f)

    out = pl.pallas_call(
        _fused_body,
        grid=(_NW + n_blocks,),
        in_specs=[
            pl.BlockSpec((chunk, in_f), lambda m: (jnp.minimum(m, _NW - 1), 0)),
            pl.BlockSpec((in_f, out_f), lambda m: (0, 0)),
            pl.BlockSpec(memory_space=pl.ANY),
            pl.BlockSpec((1, out_f), lambda m: (0, 0)),
        ],
        out_specs=pl.BlockSpec(
            (bm, out_f), lambda m: (jnp.maximum(m - _NW, 0), 0)
        ),
        out_shape=jax.ShapeDtypeStruct((n, out_f), jnp.float32),
        scratch_shapes=[
            pltpu.VMEM((n, out_f), jnp.bfloat16),
            pltpu.VMEM((2, bm, n), jnp.float32),
            pltpu.SemaphoreType.DMA((2,)),
        ],
        compiler_params=pltpu.CompilerParams(
            dimension_semantics=("arbitrary",),
        ),
    )(input, W, adj, b2)
    return out
